# baseline (speedup 1.0000x reference)
import jax
import jax.numpy as jnp
from jax import lax
from jax.experimental import pallas as pl
from jax.experimental.pallas import tpu as pltpu
from jax.experimental.pallas import tpu_sc as plsc

N_NODES = 10000
N_PAD = 10240
N_EDGES = 320000
D = 128

NC = 2
NS = 16
NW = NC * NS
CHUNK = 128
N_CHUNKS = 2560
E_PAD = N_CHUNKS * CHUNK
CHUNKS_W = N_CHUNKS // NW

ROWS_PER_TILE = N_PAD // NS

NBUF = 5
NGRP = CHUNKS_W // NBUF
DBUF = 3
DGRP = CHUNKS_W // DBUF

_mesh = plsc.VectorSubcoreMesh(core_axis_name="c", subcore_axis_name="s",
                               num_cores=NC, num_subcores=NS)


def _worker():
    c = lax.axis_index("c")
    s = lax.axis_index("s")
    wid = s * NC + c
    start_chunk = wid * CHUNKS_W
    return c, s, wid, start_chunk


def _preload_idx(idx2_hbm, idx_v, start_chunk):
    pltpu.sync_copy(idx2_hbm.at[pl.ds(start_chunk, CHUNKS_W)], idx_v)


def _sc_deg_body(dst2_hbm, out_hbm, deg_sh, didx_v, ones_v, zrow_v, ssem):
    c, s, wid, start_chunk = _worker()

    for j in range(CHUNK // 16):
        ones_v[pl.ds(j * 16, 16)] = jnp.full((16,), 1.0, jnp.float32)
        zrow_v[pl.ds(j * 16, 16)] = jnp.zeros((16,), jnp.float32)

    _preload_idx(dst2_hbm, didx_v, start_chunk)

    for j in range(ROWS_PER_TILE // CHUNK):
        pltpu.sync_copy(
            zrow_v, deg_sh.at[pl.ds(s * ROWS_PER_TILE + j * CHUNK, CHUNK)])
    plsc.subcore_barrier()

    def group(t, carry):
        for b in range(NBUF):
            j = t * NBUF + b

            @pl.when(t > 0)
            def _():
                pltpu.make_async_copy(
                    ones_v, deg_sh.at[didx_v.at[0]], ssem.at[b]).wait()

            pltpu.async_copy(ones_v, deg_sh.at[didx_v.at[j]], ssem.at[b],
                             add=True)
        return carry

    lax.fori_loop(0, NGRP, group, 0)
    for b in range(NBUF):
        pltpu.make_async_copy(ones_v, deg_sh.at[didx_v.at[0]],
                              ssem.at[b]).wait()
    plsc.subcore_barrier()
    pltpu.sync_copy(deg_sh.at[pl.ds(s * ROWS_PER_TILE, ROWS_PER_TILE)],
                    out_hbm.at[c, pl.ds(s * ROWS_PER_TILE, ROWS_PER_TILE)])


_sc_deg = pl.kernel(
    _sc_deg_body,
    out_type=jax.ShapeDtypeStruct((NC, N_PAD), jnp.float32),
    mesh=_mesh,
    scratch_types=[
        pltpu.VMEM_SHARED((N_PAD,), jnp.float32),
        pltpu.VMEM((CHUNKS_W, CHUNK), jnp.int32),
        pltpu.VMEM((CHUNK,), jnp.float32),
        pltpu.VMEM((CHUNK,), jnp.float32),
        pltpu.SemaphoreType.DMA((NBUF,)),
    ],
)


LGRP = CHUNKS_W // 4


def _sc_layer_body(g_hbm, src_hbm, dst_hbm, out_hbm,
                   acc_sh, sidx_v, didx_v, rows_v, gsem, ssem, isem):
    c, s, wid, start_chunk = _worker()

    def load_idx(j, q):
        base = (start_chunk + j) * CHUNK
        pltpu.async_copy(src_hbm.at[pl.ds(base, CHUNK)], sidx_v.at[q],
                         isem.at[q])
        pltpu.async_copy(dst_hbm.at[pl.ds(base, CHUNK)], didx_v.at[q],
                         isem.at[q])

    def wait_idx(q):
        pltpu.make_async_copy(src_hbm.at[pl.ds(0, CHUNK)], sidx_v.at[q],
                              isem.at[q]).wait()
        pltpu.make_async_copy(dst_hbm.at[pl.ds(0, CHUNK)], didx_v.at[q],
                              isem.at[q]).wait()

    load_idx(0, 0)
    load_idx(1, 1)
    pltpu.sync_copy(g_hbm.at[pl.ds(s * ROWS_PER_TILE, ROWS_PER_TILE)],
                    acc_sh.at[pl.ds(s * ROWS_PER_TILE, ROWS_PER_TILE)])
    plsc.subcore_barrier()

    def group(t, carry):
        for k in range(4):
            b = k % 2
            bp = (k + 1) % 2
            q = k
            q2 = (k + 2) % 4
            qp = (k + 3) % 4
            j = t * 4 + k

            if k < 2:
                @pl.when(t > 0)
                def _():
                    pltpu.make_async_copy(rows_v.at[b],
                                          acc_sh.at[didx_v.at[0]],
                                          ssem.at[b]).wait()
            else:
                pltpu.make_async_copy(rows_v.at[b], acc_sh.at[didx_v.at[0]],
                                      ssem.at[b]).wait()
            wait_idx(q)
            pltpu.async_copy(g_hbm.at[sidx_v.at[q]], rows_v.at[b], gsem.at[b])

            @pl.when(j + 2 < CHUNKS_W)
            def _():
                load_idx(j + 2, q2)

            if k == 0:
                @pl.when(t > 0)
                def _():
                    pltpu.make_async_copy(g_hbm.at[sidx_v.at[qp]],
                                          rows_v.at[bp], gsem.at[bp]).wait()
                    pltpu.async_copy(rows_v.at[bp], acc_sh.at[didx_v.at[qp]],
                                     ssem.at[bp], add=True)
            else:
                pltpu.make_async_copy(g_hbm.at[sidx_v.at[qp]],
                                      rows_v.at[bp], gsem.at[bp]).wait()
                pltpu.async_copy(rows_v.at[bp], acc_sh.at[didx_v.at[qp]],
                                 ssem.at[bp], add=True)
        return carry

    lax.fori_loop(0, LGRP, group, 0)
    pltpu.make_async_copy(g_hbm.at[sidx_v.at[3]], rows_v.at[1],
                          gsem.at[1]).wait()
    pltpu.async_copy(rows_v.at[1], acc_sh.at[didx_v.at[3]],
                     ssem.at[1], add=True)
    for b in range(2):
        pltpu.make_async_copy(rows_v.at[b], acc_sh.at[didx_v.at[0]],
                              ssem.at[b]).wait()
    plsc.subcore_barrier()
    pltpu.sync_copy(acc_sh.at[pl.ds(s * ROWS_PER_TILE, ROWS_PER_TILE)],
                    out_hbm.at[c, pl.ds(s * ROWS_PER_TILE, ROWS_PER_TILE)])


_sc_layer = pl.kernel(
    _sc_layer_body,
    out_type=jax.ShapeDtypeStruct((NC, N_PAD, D), jnp.float32),
    mesh=_mesh,
    scratch_types=[
        pltpu.VMEM_SHARED((N_PAD, D), jnp.float32),
        pltpu.VMEM((4, CHUNK), jnp.int32),
        pltpu.VMEM((4, CHUNK), jnp.int32),
        pltpu.VMEM((2, CHUNK, D), jnp.float32),
        pltpu.SemaphoreType.DMA((2,)),
        pltpu.SemaphoreType.DMA((2,)),
        pltpu.SemaphoreType.DMA((4,)),
    ],
)


def _sc_decode_body(z_hbm, src2_hbm, dst2_hbm, out_hbm,
                    sidx_v, didx_v, za_v, zb_v, scr_v, out_v, gsem, osem):
    c, s, wid, start_chunk = _worker()
    lane = lax.broadcasted_iota(jnp.int32, (16,), 0)
    lane16 = lane * 16

    _preload_idx(src2_hbm, sidx_v, start_chunk)
    _preload_idx(dst2_hbm, didx_v, start_chunk)

    def compute(b, j):
        @plsc.parallel_loop(0, CHUNK // 16)
        def blk_body(k):
            for e in range(16):
                va_r = za_v.at[b].at[k * 16 + e]
                vb_r = zb_v.at[b].at[k * 16 + e]
                acc = jnp.zeros((16,), jnp.float32)
                for q in range(D // 16):
                    acc = acc + (va_r[pl.ds(q * 16, 16)]
                                 * vb_r[pl.ds(q * 16, 16)])
                scr_v[pl.ds(k * 256 + e * 16, 16)] = acc
            tot = jnp.zeros((16,), jnp.float32)
            for l in range(16):
                col = plsc.load_gather(scr_v, [k * 256 + lane16 + l])
                tot = tot + col
            prob = 1.0 / (1.0 + jnp.exp(-tot))
            out_v[b, pl.ds(k * 16, 16)] = prob

    def gathers(j, b):
        pltpu.async_copy(z_hbm.at[sidx_v.at[j]], za_v.at[b], gsem.at[b])
        pltpu.async_copy(z_hbm.at[didx_v.at[j]], zb_v.at[b], gsem.at[b])

    def wait_gathers(j, b):
        pltpu.make_async_copy(z_hbm.at[sidx_v.at[j]], za_v.at[b],
                              gsem.at[b]).wait()
        pltpu.make_async_copy(z_hbm.at[didx_v.at[j]], zb_v.at[b],
                              gsem.at[b]).wait()

    def wait_out(b):
        pltpu.make_async_copy(out_v.at[b], out_hbm.at[pl.ds(0, CHUNK)],
                              osem.at[b]).wait()

    def store_out(j, b):
        base = (start_chunk + j) * CHUNK
        pltpu.async_copy(out_v.at[b], out_hbm.at[pl.ds(base, CHUNK)],
                         osem.at[b])

    for b in range(DBUF):
        gathers(b, b)

    def group(t, carry):
        for b in range(DBUF):
            j = t * DBUF + b
            wait_gathers(j, b)

            @pl.when(t > 0)
            def _():
                wait_out(b)

            compute(b, j)
            store_out(j, b)

            @pl.when(j + DBUF < CHUNKS_W)
            def _():
                gathers(j + DBUF, b)
        return carry

    lax.fori_loop(0, DGRP, group, 0)
    for b in range(CHUNKS_W - DBUF * DGRP):
        j = DBUF * DGRP + b
        wait_gathers(j, b)
        wait_out(b)
        compute(b, j)
        store_out(j, b)
    for b in range(DBUF):
        wait_out(b)


_sc_decode = pl.kernel(
    _sc_decode_body,
    out_type=jax.ShapeDtypeStruct((E_PAD,), jnp.float32),
    mesh=_mesh,
    compiler_params=pltpu.CompilerParams(needs_layout_passes=False),
    scratch_types=[
        pltpu.VMEM((CHUNKS_W, CHUNK), jnp.int32),
        pltpu.VMEM((CHUNKS_W, CHUNK), jnp.int32),
        pltpu.VMEM((DBUF, CHUNK, D), jnp.float32),
        pltpu.VMEM((DBUF, CHUNK, D), jnp.float32),
        pltpu.VMEM((CHUNK // 16 * 256,), jnp.float32),
        pltpu.VMEM((DBUF, CHUNK), jnp.float32),
        pltpu.SemaphoreType.DMA((DBUF,)),
        pltpu.SemaphoreType.DMA((DBUF,)),
    ],
)


ROW_BLK = 1024
GRID = N_PAD // ROW_BLK


def _tc_mm_scale_body(deg0_ref, deg1_ref, x_ref, w_ref, g_ref, dis_ref):
    deg = deg0_ref[...] + deg1_ref[...] + 1.0
    dis = lax.rsqrt(deg)
    m = jnp.dot(x_ref[...], w_ref[...], preferred_element_type=jnp.float32)
    g_ref[...] = m * dis
    dis_ref[...] = dis


def _tc_mm_scale(deg0, deg1, x, w):
    return pl.pallas_call(
        _tc_mm_scale_body,
        grid=(GRID,),
        in_specs=[
            pl.BlockSpec((ROW_BLK, 1), lambda i: (i, 0)),
            pl.BlockSpec((ROW_BLK, 1), lambda i: (i, 0)),
            pl.BlockSpec((ROW_BLK, D), lambda i: (i, 0)),
            pl.BlockSpec((D, D), lambda i: (0, 0)),
        ],
        out_specs=[
            pl.BlockSpec((ROW_BLK, D), lambda i: (i, 0)),
            pl.BlockSpec((ROW_BLK, 1), lambda i: (i, 0)),
        ],
        out_shape=[
            jax.ShapeDtypeStruct((N_PAD, D), jnp.float32),
            jax.ShapeDtypeStruct((N_PAD, 1), jnp.float32),
        ],
    )(deg0, deg1, x, w)


def _tc_mid_body(p0_ref, p1_ref, gp_ref, dis_ref, b_ref, w_ref, g_ref):
    dis = dis_ref[...]
    agg = (p0_ref[...] + p1_ref[...] - gp_ref[...]) * dis
    h = jnp.maximum(agg + b_ref[...], 0.0)
    g_ref[...] = jnp.dot(h, w_ref[...], preferred_element_type=jnp.float32) * dis


def _tc_mid(p0, p1, gp, dis, b, w):
    return pl.pallas_call(
        _tc_mid_body,
        grid=(GRID,),
        in_specs=[
            pl.BlockSpec((ROW_BLK, D), lambda i: (i, 0)),
            pl.BlockSpec((ROW_BLK, D), lambda i: (i, 0)),
            pl.BlockSpec((ROW_BLK, D), lambda i: (i, 0)),
            pl.BlockSpec((ROW_BLK, 1), lambda i: (i, 0)),
            pl.BlockSpec((1, D), lambda i: (0, 0)),
            pl.BlockSpec((D, D), lambda i: (0, 0)),
        ],
        out_specs=pl.BlockSpec((ROW_BLK, D), lambda i: (i, 0)),
        out_shape=jax.ShapeDtypeStruct((N_PAD, D), jnp.float32),
    )(p0, p1, gp, dis, b, w)


def _tc_z_body(p0_ref, p1_ref, gp_ref, dis_ref, b_ref, z_ref):
    z_ref[...] = ((p0_ref[...] + p1_ref[...] - gp_ref[...]) * dis_ref[...]
                  + b_ref[...])


def _tc_z(p0, p1, gp, dis, b):
    return pl.pallas_call(
        _tc_z_body,
        grid=(GRID,),
        in_specs=[
            pl.BlockSpec((ROW_BLK, D), lambda i: (i, 0)),
            pl.BlockSpec((ROW_BLK, D), lambda i: (i, 0)),
            pl.BlockSpec((ROW_BLK, D), lambda i: (i, 0)),
            pl.BlockSpec((ROW_BLK, 1), lambda i: (i, 0)),
            pl.BlockSpec((1, D), lambda i: (0, 0)),
        ],
        out_specs=pl.BlockSpec((ROW_BLK, D), lambda i: (i, 0)),
        out_shape=jax.ShapeDtypeStruct((N_PAD, D), jnp.float32),
    )(p0, p1, gp, dis, b)


def kernel(x, edge_index, W1, b1, W2, b2, W3, b3):
    n_pad_e = E_PAD - N_EDGES
    pad_idx = N_NODES + (jnp.arange(n_pad_e, dtype=jnp.int32)
                         % (N_PAD - N_NODES))
    src = jnp.concatenate([edge_index[0], pad_idx])
    dst = jnp.concatenate([edge_index[1], pad_idx])
    src2 = src.reshape(N_CHUNKS, CHUNK)
    dst2 = dst.reshape(N_CHUNKS, CHUNK)

    degp = _sc_deg(dst2)
    deg0 = degp[0].reshape(N_PAD, 1)
    deg1 = degp[1].reshape(N_PAD, 1)

    xp = jnp.zeros((N_PAD, D), jnp.float32).at[:N_NODES].set(x)
    g1, dis = _tc_mm_scale(deg0, deg1, xp, W1)
    p1 = _sc_layer(g1, src, dst)
    g2 = _tc_mid(p1[0], p1[1], g1, dis, b1.reshape(1, D), W2)
    p2 = _sc_layer(g2, src, dst)
    g3 = _tc_mid(p2[0], p2[1], g2, dis, b2.reshape(1, D), W3)
    p3 = _sc_layer(g3, src, dst)
    z = _tc_z(p3[0], p3[1], g3, dis, b3.reshape(1, D))
    return _sc_decode(z, src2, dst2)[:N_EDGES]


# speedup vs baseline: 1.0153x; 1.0153x over previous
import jax
import jax.numpy as jnp
from jax import lax
from jax.experimental import pallas as pl
from jax.experimental.pallas import tpu as pltpu
from jax.experimental.pallas import tpu_sc as plsc

N_NODES = 10000
N_PAD = 10240
N_EDGES = 320000
D = 128

NC = 2
NS = 16
NW = NC * NS
CHUNK = 128
N_CHUNKS = 2560
E_PAD = N_CHUNKS * CHUNK
CHUNKS_W = N_CHUNKS // NW

ROWS_PER_TILE = N_PAD // NS

NBUF = 5
NGRP = CHUNKS_W // NBUF
DBUF = 2
DGRP = CHUNKS_W // DBUF

_mesh = plsc.VectorSubcoreMesh(core_axis_name="c", subcore_axis_name="s",
                               num_cores=NC, num_subcores=NS)


def _worker():
    c = lax.axis_index("c")
    s = lax.axis_index("s")
    wid = s * NC + c
    start_chunk = wid * CHUNKS_W
    return c, s, wid, start_chunk


def _preload_idx(idx2_hbm, idx_v, start_chunk):
    pltpu.sync_copy(idx2_hbm.at[pl.ds(start_chunk, CHUNKS_W)], idx_v)


def _sc_deg_body(dst2_hbm, out_hbm, deg_sh, didx_v, ones_v, zrow_v, ssem):
    c, s, wid, start_chunk = _worker()

    for j in range(CHUNK // 16):
        ones_v[pl.ds(j * 16, 16)] = jnp.full((16,), 1.0, jnp.float32)
        zrow_v[pl.ds(j * 16, 16)] = jnp.zeros((16,), jnp.float32)

    _preload_idx(dst2_hbm, didx_v, start_chunk)

    for j in range(ROWS_PER_TILE // CHUNK):
        pltpu.sync_copy(
            zrow_v, deg_sh.at[pl.ds(s * ROWS_PER_TILE + j * CHUNK, CHUNK)])
    plsc.subcore_barrier()

    def group(t, carry):
        for b in range(NBUF):
            j = t * NBUF + b

            @pl.when(t > 0)
            def _():
                pltpu.make_async_copy(
                    ones_v, deg_sh.at[didx_v.at[0]], ssem.at[b]).wait()

            pltpu.async_copy(ones_v, deg_sh.at[didx_v.at[j]], ssem.at[b],
                             add=True)
        return carry

    lax.fori_loop(0, NGRP, group, 0)
    for b in range(NBUF):
        pltpu.make_async_copy(ones_v, deg_sh.at[didx_v.at[0]],
                              ssem.at[b]).wait()
    plsc.subcore_barrier()
    pltpu.sync_copy(deg_sh.at[pl.ds(s * ROWS_PER_TILE, ROWS_PER_TILE)],
                    out_hbm.at[c, pl.ds(s * ROWS_PER_TILE, ROWS_PER_TILE)])


_sc_deg = pl.kernel(
    _sc_deg_body,
    out_type=jax.ShapeDtypeStruct((NC, N_PAD), jnp.float32),
    mesh=_mesh,
    scratch_types=[
        pltpu.VMEM_SHARED((N_PAD,), jnp.float32),
        pltpu.VMEM((CHUNKS_W, CHUNK), jnp.int32),
        pltpu.VMEM((CHUNK,), jnp.float32),
        pltpu.VMEM((CHUNK,), jnp.float32),
        pltpu.SemaphoreType.DMA((NBUF,)),
    ],
)


LGRP = CHUNKS_W // 4


def _sc_layer_body(g_hbm, src_hbm, dst3_hbm, out_hbm,
                   acc_sh, sidx_v, didx_v, rows_v, gsem, ssem, isem):
    c, s, wid, start_chunk = _worker()

    def load_idx(j, q):
        base = (start_chunk + j) * CHUNK
        pltpu.async_copy(src_hbm.at[pl.ds(base, CHUNK)], sidx_v.at[q],
                         isem.at[q])
        pltpu.async_copy(dst3_hbm.at[start_chunk + j], didx_v.at[q],
                         isem.at[q])

    def wait_idx(q):
        pltpu.make_async_copy(src_hbm.at[pl.ds(0, CHUNK)], sidx_v.at[q],
                              isem.at[q]).wait()
        pltpu.make_async_copy(dst3_hbm.at[start_chunk], didx_v.at[q],
                              isem.at[q]).wait()

    def scatter(bb, qq):
        for h in range(2):
            pltpu.async_copy(rows_v.at[bb].at[pl.ds(h * 64, 64)],
                             acc_sh.at[didx_v.at[qq, h]],
                             ssem.at[bb], add=True)

    def wait_scatter(bb):
        for _ in range(2):
            pltpu.make_async_copy(rows_v.at[bb].at[pl.ds(0, 64)],
                                  acc_sh.at[didx_v.at[0, 0]],
                                  ssem.at[bb]).wait()

    load_idx(0, 0)
    load_idx(1, 1)
    pltpu.sync_copy(g_hbm.at[pl.ds(s * ROWS_PER_TILE, ROWS_PER_TILE)],
                    acc_sh.at[pl.ds(s * ROWS_PER_TILE, ROWS_PER_TILE)])
    plsc.subcore_barrier()

    def group(t, carry):
        for k in range(4):
            b = k % 2
            bp = (k + 1) % 2
            q = k
            q2 = (k + 2) % 4
            qp = (k + 3) % 4
            j = t * 4 + k

            if k < 2:
                @pl.when(t > 0)
                def _():
                    wait_scatter(b)
            else:
                wait_scatter(b)
            wait_idx(q)
            pltpu.async_copy(g_hbm.at[sidx_v.at[q]], rows_v.at[b], gsem.at[b])

            @pl.when(j + 2 < CHUNKS_W)
            def _():
                load_idx(j + 2, q2)

            if k == 0:
                @pl.when(t > 0)
                def _():
                    pltpu.make_async_copy(g_hbm.at[sidx_v.at[qp]],
                                          rows_v.at[bp], gsem.at[bp]).wait()
                    scatter(bp, qp)
            else:
                pltpu.make_async_copy(g_hbm.at[sidx_v.at[qp]],
                                      rows_v.at[bp], gsem.at[bp]).wait()
                scatter(bp, qp)
        return carry

    lax.fori_loop(0, LGRP, group, 0)
    pltpu.make_async_copy(g_hbm.at[sidx_v.at[3]], rows_v.at[1],
                          gsem.at[1]).wait()
    scatter(1, 3)
    for b in range(2):
        wait_scatter(b)
    plsc.subcore_barrier()
    pltpu.sync_copy(acc_sh.at[pl.ds(s * ROWS_PER_TILE, ROWS_PER_TILE)],
                    out_hbm.at[c, pl.ds(s * ROWS_PER_TILE, ROWS_PER_TILE)])


_sc_layer = pl.kernel(
    _sc_layer_body,
    out_type=jax.ShapeDtypeStruct((NC, N_PAD, D), jnp.float32),
    mesh=_mesh,
    scratch_types=[
        pltpu.VMEM_SHARED((N_PAD, D), jnp.float32),
        pltpu.VMEM((4, CHUNK), jnp.int32),
        pltpu.VMEM((4, 2, 64), jnp.int32),
        pltpu.VMEM((2, CHUNK, D), jnp.float32),
        pltpu.SemaphoreType.DMA((2,)),
        pltpu.SemaphoreType.DMA((2,)),
        pltpu.SemaphoreType.DMA((4,)),
    ],
)


def _sc_decode_body(z_hbm, src2_hbm, dst2_hbm, out_hbm,
                    sidx_v, didx_v, za_v, zb_v, scr_v, out_v, gsem, osem):
    c, s, wid, start_chunk = _worker()
    lane = lax.broadcasted_iota(jnp.int32, (16,), 0)
    lane16 = lane * 16

    _preload_idx(src2_hbm, sidx_v, start_chunk)
    _preload_idx(dst2_hbm, didx_v, start_chunk)

    def compute(b, j):
        @plsc.parallel_loop(0, CHUNK // 16)
        def blk_body(k):
            for e in range(16):
                va_r = za_v.at[b].at[k * 16 + e]
                vb_r = zb_v.at[b].at[k * 16 + e]
                acc = jnp.zeros((16,), jnp.float32)
                for q in range(D // 16):
                    acc = acc + (va_r[pl.ds(q * 16, 16)]
                                 * vb_r[pl.ds(q * 16, 16)])
                scr_v[pl.ds(k * 256 + e * 16, 16)] = acc
            tot = jnp.zeros((16,), jnp.float32)
            for l in range(16):
                col = plsc.load_gather(scr_v, [k * 256 + lane16 + l])
                tot = tot + col
            prob = 1.0 / (1.0 + jnp.exp(-tot))
            out_v[b, pl.ds(k * 16, 16)] = prob

    def gathers(j, b):
        pltpu.async_copy(z_hbm.at[sidx_v.at[j]], za_v.at[b], gsem.at[b])
        pltpu.async_copy(z_hbm.at[didx_v.at[j]], zb_v.at[b], gsem.at[b])

    def wait_gathers(j, b):
        pltpu.make_async_copy(z_hbm.at[sidx_v.at[j]], za_v.at[b],
                              gsem.at[b]).wait()
        pltpu.make_async_copy(z_hbm.at[didx_v.at[j]], zb_v.at[b],
                              gsem.at[b]).wait()

    def wait_out(b):
        pltpu.make_async_copy(out_v.at[b], out_hbm.at[pl.ds(0, CHUNK)],
                              osem.at[b]).wait()

    def store_out(j, b):
        base = (start_chunk + j) * CHUNK
        pltpu.async_copy(out_v.at[b], out_hbm.at[pl.ds(base, CHUNK)],
                         osem.at[b])

    for b in range(DBUF):
        gathers(b, b)

    def group(t, carry):
        for b in range(DBUF):
            j = t * DBUF + b
            wait_gathers(j, b)

            @pl.when(t > 0)
            def _():
                wait_out(b)

            compute(b, j)
            store_out(j, b)

            @pl.when(j + DBUF < CHUNKS_W)
            def _():
                gathers(j + DBUF, b)
        return carry

    lax.fori_loop(0, DGRP, group, 0)
    for b in range(CHUNKS_W - DBUF * DGRP):
        j = DBUF * DGRP + b
        wait_gathers(j, b)
        wait_out(b)
        compute(b, j)
        store_out(j, b)
    for b in range(DBUF):
        wait_out(b)


_sc_decode = pl.kernel(
    _sc_decode_body,
    out_type=jax.ShapeDtypeStruct((E_PAD,), jnp.float32),
    mesh=_mesh,
    compiler_params=pltpu.CompilerParams(needs_layout_passes=False),
    scratch_types=[
        pltpu.VMEM((CHUNKS_W, CHUNK), jnp.int32),
        pltpu.VMEM((CHUNKS_W, CHUNK), jnp.int32),
        pltpu.VMEM((DBUF, CHUNK, D), jnp.float32),
        pltpu.VMEM((DBUF, CHUNK, D), jnp.float32),
        pltpu.VMEM((CHUNK // 16 * 256,), jnp.float32),
        pltpu.VMEM((DBUF, CHUNK), jnp.float32),
        pltpu.SemaphoreType.DMA((DBUF,)),
        pltpu.SemaphoreType.DMA((DBUF,)),
    ],
)


ROW_BLK = 1024
GRID = N_PAD // ROW_BLK


def _tc_mm_scale_body(deg0_ref, deg1_ref, x_ref, w_ref, g_ref, dis_ref):
    deg = deg0_ref[...] + deg1_ref[...] + 1.0
    dis = lax.rsqrt(deg)
    m = jnp.dot(x_ref[...], w_ref[...], preferred_element_type=jnp.float32)
    g_ref[...] = m * dis
    dis_ref[...] = dis


def _tc_mm_scale(deg0, deg1, x, w):
    return pl.pallas_call(
        _tc_mm_scale_body,
        grid=(GRID,),
        in_specs=[
            pl.BlockSpec((ROW_BLK, 1), lambda i: (i, 0)),
            pl.BlockSpec((ROW_BLK, 1), lambda i: (i, 0)),
            pl.BlockSpec((ROW_BLK, D), lambda i: (i, 0)),
            pl.BlockSpec((D, D), lambda i: (0, 0)),
        ],
        out_specs=[
            pl.BlockSpec((ROW_BLK, D), lambda i: (i, 0)),
            pl.BlockSpec((ROW_BLK, 1), lambda i: (i, 0)),
        ],
        out_shape=[
            jax.ShapeDtypeStruct((N_PAD, D), jnp.float32),
            jax.ShapeDtypeStruct((N_PAD, 1), jnp.float32),
        ],
    )(deg0, deg1, x, w)


def _tc_mid_body(p0_ref, p1_ref, gp_ref, dis_ref, b_ref, w_ref, g_ref):
    dis = dis_ref[...]
    agg = (p0_ref[...] + p1_ref[...] - gp_ref[...]) * dis
    h = jnp.maximum(agg + b_ref[...], 0.0)
    g_ref[...] = jnp.dot(h, w_ref[...], preferred_element_type=jnp.float32) * dis


def _tc_mid(p0, p1, gp, dis, b, w):
    return pl.pallas_call(
        _tc_mid_body,
        grid=(GRID,),
        in_specs=[
            pl.BlockSpec((ROW_BLK, D), lambda i: (i, 0)),
            pl.BlockSpec((ROW_BLK, D), lambda i: (i, 0)),
            pl.BlockSpec((ROW_BLK, D), lambda i: (i, 0)),
            pl.BlockSpec((ROW_BLK, 1), lambda i: (i, 0)),
            pl.BlockSpec((1, D), lambda i: (0, 0)),
            pl.BlockSpec((D, D), lambda i: (0, 0)),
        ],
        out_specs=pl.BlockSpec((ROW_BLK, D), lambda i: (i, 0)),
        out_shape=jax.ShapeDtypeStruct((N_PAD, D), jnp.float32),
    )(p0, p1, gp, dis, b, w)


def _tc_z_body(p0_ref, p1_ref, gp_ref, dis_ref, b_ref, z_ref):
    z_ref[...] = ((p0_ref[...] + p1_ref[...] - gp_ref[...]) * dis_ref[...]
                  + b_ref[...])


def _tc_z(p0, p1, gp, dis, b):
    return pl.pallas_call(
        _tc_z_body,
        grid=(GRID,),
        in_specs=[
            pl.BlockSpec((ROW_BLK, D), lambda i: (i, 0)),
            pl.BlockSpec((ROW_BLK, D), lambda i: (i, 0)),
            pl.BlockSpec((ROW_BLK, D), lambda i: (i, 0)),
            pl.BlockSpec((ROW_BLK, 1), lambda i: (i, 0)),
            pl.BlockSpec((1, D), lambda i: (0, 0)),
        ],
        out_specs=pl.BlockSpec((ROW_BLK, D), lambda i: (i, 0)),
        out_shape=jax.ShapeDtypeStruct((N_PAD, D), jnp.float32),
    )(p0, p1, gp, dis, b)


def kernel(x, edge_index, W1, b1, W2, b2, W3, b3):
    n_pad_e = E_PAD - N_EDGES
    pad_idx = N_NODES + (jnp.arange(n_pad_e, dtype=jnp.int32)
                         % (N_PAD - N_NODES))
    src = jnp.concatenate([edge_index[0], pad_idx])
    dst = jnp.concatenate([edge_index[1], pad_idx])
    src2 = src.reshape(N_CHUNKS, CHUNK)
    dst2 = dst.reshape(N_CHUNKS, CHUNK)
    dst3 = dst.reshape(N_CHUNKS, 2, 64)

    degp = _sc_deg(dst2)
    deg0 = degp[0].reshape(N_PAD, 1)
    deg1 = degp[1].reshape(N_PAD, 1)

    xp = jnp.zeros((N_PAD, D), jnp.float32).at[:N_NODES].set(x)
    g1, dis = _tc_mm_scale(deg0, deg1, xp, W1)
    p1 = _sc_layer(g1, src, dst3)
    g2 = _tc_mid(p1[0], p1[1], g1, dis, b1.reshape(1, D), W2)
    p2 = _sc_layer(g2, src, dst3)
    g3 = _tc_mid(p2[0], p2[1], g2, dis, b2.reshape(1, D), W3)
    p3 = _sc_layer(g3, src, dst3)
    z = _tc_z(p3[0], p3[1], g3, dis, b3.reshape(1, D))
    return _sc_decode(z, src2, dst2)[:N_EDGES]


# speedup vs baseline: 1.0524x; 1.0366x over previous
import jax
import jax.numpy as jnp
from jax import lax
from jax.experimental import pallas as pl
from jax.experimental.pallas import tpu as pltpu
from jax.experimental.pallas import tpu_sc as plsc

N_NODES = 10000
N_PAD = 10240
N_EDGES = 320000
D = 128

NC = 2
NS = 16
NW = NC * NS
CHUNK = 128
N_CHUNKS = 2560
E_PAD = N_CHUNKS * CHUNK
CHUNKS_W = N_CHUNKS // NW

ROWS_PER_TILE = N_PAD // NS

NBUF = 5
NGRP = CHUNKS_W // NBUF
DBUF = 2
DGRP = CHUNKS_W // DBUF

_mesh = plsc.VectorSubcoreMesh(core_axis_name="c", subcore_axis_name="s",
                               num_cores=NC, num_subcores=NS)


def _worker():
    c = lax.axis_index("c")
    s = lax.axis_index("s")
    wid = s * NC + c
    start_chunk = wid * CHUNKS_W
    return c, s, wid, start_chunk


def _preload_idx(idx2_hbm, idx_v, start_chunk):
    pltpu.sync_copy(idx2_hbm.at[pl.ds(start_chunk, CHUNKS_W)], idx_v)


def _sc_deg_body(dst2_hbm, out_hbm, deg_sh, didx_v, ones_v, zrow_v, ssem):
    c, s, wid, start_chunk = _worker()

    for j in range(CHUNK // 16):
        ones_v[pl.ds(j * 16, 16)] = jnp.full((16,), 1.0, jnp.float32)
        zrow_v[pl.ds(j * 16, 16)] = jnp.zeros((16,), jnp.float32)

    _preload_idx(dst2_hbm, didx_v, start_chunk)

    for j in range(ROWS_PER_TILE // CHUNK):
        pltpu.sync_copy(
            zrow_v, deg_sh.at[pl.ds(s * ROWS_PER_TILE + j * CHUNK, CHUNK)])
    plsc.subcore_barrier()

    def group(t, carry):
        for b in range(NBUF):
            j = t * NBUF + b

            @pl.when(t > 0)
            def _():
                pltpu.make_async_copy(
                    ones_v, deg_sh.at[didx_v.at[0]], ssem.at[b]).wait()

            pltpu.async_copy(ones_v, deg_sh.at[didx_v.at[j]], ssem.at[b],
                             add=True)
        return carry

    lax.fori_loop(0, NGRP, group, 0)
    for b in range(NBUF):
        pltpu.make_async_copy(ones_v, deg_sh.at[didx_v.at[0]],
                              ssem.at[b]).wait()
    plsc.subcore_barrier()
    pltpu.sync_copy(deg_sh.at[pl.ds(s * ROWS_PER_TILE, ROWS_PER_TILE)],
                    out_hbm.at[c, pl.ds(s * ROWS_PER_TILE, ROWS_PER_TILE)])


_sc_deg = pl.kernel(
    _sc_deg_body,
    out_type=jax.ShapeDtypeStruct((NC, N_PAD), jnp.float32),
    mesh=_mesh,
    scratch_types=[
        pltpu.VMEM_SHARED((N_PAD,), jnp.float32),
        pltpu.VMEM((CHUNKS_W, CHUNK), jnp.int32),
        pltpu.VMEM((CHUNK,), jnp.float32),
        pltpu.VMEM((CHUNK,), jnp.float32),
        pltpu.SemaphoreType.DMA((NBUF,)),
    ],
)


LGRP = CHUNKS_W // 4


def _sc_layer_body(g_hbm, src_hbm, dst3_hbm, out_hbm,
                   acc_sh, sidx_v, didx_v, rows_v, gsem, ssem, isem):
    c, s, wid, start_chunk = _worker()

    def load_idx(j, q):
        base = (start_chunk + j) * CHUNK
        pltpu.async_copy(src_hbm.at[pl.ds(base, CHUNK)], sidx_v.at[q],
                         isem.at[q])
        pltpu.async_copy(dst3_hbm.at[start_chunk + j], didx_v.at[q],
                         isem.at[q])

    def wait_idx(q):
        pltpu.make_async_copy(src_hbm.at[pl.ds(0, CHUNK)], sidx_v.at[q],
                              isem.at[q]).wait()
        pltpu.make_async_copy(dst3_hbm.at[start_chunk], didx_v.at[q],
                              isem.at[q]).wait()

    def scatter(bb, qq):
        for h in range(2):
            pltpu.async_copy(rows_v.at[bb].at[pl.ds(h * 64, 64)],
                             acc_sh.at[didx_v.at[qq, h]],
                             ssem.at[bb], add=True)

    def wait_scatter(bb):
        for _ in range(2):
            pltpu.make_async_copy(rows_v.at[bb].at[pl.ds(0, 64)],
                                  acc_sh.at[didx_v.at[0, 0]],
                                  ssem.at[bb]).wait()

    load_idx(0, 0)
    load_idx(1, 1)
    pltpu.sync_copy(g_hbm.at[pl.ds(s * ROWS_PER_TILE, ROWS_PER_TILE)],
                    acc_sh.at[pl.ds(s * ROWS_PER_TILE, ROWS_PER_TILE)])
    plsc.subcore_barrier()

    def group(t, carry):
        for k in range(4):
            b = k % 2
            bp = (k + 1) % 2
            q = k
            q2 = (k + 2) % 4
            qp = (k + 3) % 4
            j = t * 4 + k

            if k < 2:
                @pl.when(t > 0)
                def _():
                    wait_scatter(b)
            else:
                wait_scatter(b)
            wait_idx(q)
            pltpu.async_copy(g_hbm.at[sidx_v.at[q]], rows_v.at[b], gsem.at[b])

            @pl.when(j + 2 < CHUNKS_W)
            def _():
                load_idx(j + 2, q2)

            if k == 0:
                @pl.when(t > 0)
                def _():
                    pltpu.make_async_copy(g_hbm.at[sidx_v.at[qp]],
                                          rows_v.at[bp], gsem.at[bp]).wait()
                    scatter(bp, qp)
            else:
                pltpu.make_async_copy(g_hbm.at[sidx_v.at[qp]],
                                      rows_v.at[bp], gsem.at[bp]).wait()
                scatter(bp, qp)
        return carry

    lax.fori_loop(0, LGRP, group, 0)
    pltpu.make_async_copy(g_hbm.at[sidx_v.at[3]], rows_v.at[1],
                          gsem.at[1]).wait()
    scatter(1, 3)
    for b in range(2):
        wait_scatter(b)
    plsc.subcore_barrier()
    pltpu.sync_copy(acc_sh.at[pl.ds(s * ROWS_PER_TILE, ROWS_PER_TILE)],
                    out_hbm.at[c, pl.ds(s * ROWS_PER_TILE, ROWS_PER_TILE)])


_sc_layer = pl.kernel(
    _sc_layer_body,
    out_type=jax.ShapeDtypeStruct((NC, N_PAD, D), jnp.float32),
    mesh=_mesh,
    scratch_types=[
        pltpu.VMEM_SHARED((N_PAD, D), jnp.float32),
        pltpu.VMEM((4, CHUNK), jnp.int32),
        pltpu.VMEM((4, 2, 64), jnp.int32),
        pltpu.VMEM((2, CHUNK, D), jnp.float32),
        pltpu.SemaphoreType.DMA((2,)),
        pltpu.SemaphoreType.DMA((2,)),
        pltpu.SemaphoreType.DMA((4,)),
    ],
)


def _sc_decode_body(z_hbm, src2_hbm, dst2_hbm, out_hbm,
                    sidx_v, didx_v, za_v, zb_v, scr_v, out_v, gsem, osem):
    c, s, wid, start_chunk = _worker()
    lane = lax.broadcasted_iota(jnp.int32, (16,), 0)
    lane16 = lane * 16

    _preload_idx(src2_hbm, sidx_v, start_chunk)
    _preload_idx(dst2_hbm, didx_v, start_chunk)

    def compute(b, j):
        @plsc.parallel_loop(0, CHUNK // 16, unroll=2)
        def blk_body(k):
            for e in range(16):
                va_r = za_v.at[b].at[k * 16 + e]
                vb_r = zb_v.at[b].at[k * 16 + e]
                acc0 = jnp.zeros((16,), jnp.float32)
                acc1 = jnp.zeros((16,), jnp.float32)
                for q in range(0, D // 16, 2):
                    acc0 = acc0 + (va_r[pl.ds(q * 16, 16)]
                                   * vb_r[pl.ds(q * 16, 16)])
                    acc1 = acc1 + (va_r[pl.ds((q + 1) * 16, 16)]
                                   * vb_r[pl.ds((q + 1) * 16, 16)])
                scr_v[pl.ds(k * 256 + e * 16, 16)] = acc0 + acc1
            tot = jnp.zeros((16,), jnp.float32)
            for l in range(16):
                col = plsc.load_gather(scr_v, [k * 256 + lane16 + l])
                tot = tot + col
            prob = 1.0 / (1.0 + jnp.exp(-tot))
            out_v[b, pl.ds(k * 16, 16)] = prob

    def gathers(j, b):
        pltpu.async_copy(z_hbm.at[sidx_v.at[j]], za_v.at[b], gsem.at[b])
        pltpu.async_copy(z_hbm.at[didx_v.at[j]], zb_v.at[b], gsem.at[b])

    def wait_gathers(j, b):
        pltpu.make_async_copy(z_hbm.at[sidx_v.at[j]], za_v.at[b],
                              gsem.at[b]).wait()
        pltpu.make_async_copy(z_hbm.at[didx_v.at[j]], zb_v.at[b],
                              gsem.at[b]).wait()

    def wait_out(b):
        pltpu.make_async_copy(out_v.at[b], out_hbm.at[pl.ds(0, CHUNK)],
                              osem.at[b]).wait()

    def store_out(j, b):
        base = (start_chunk + j) * CHUNK
        pltpu.async_copy(out_v.at[b], out_hbm.at[pl.ds(base, CHUNK)],
                         osem.at[b])

    for b in range(DBUF):
        gathers(b, b)

    def group(t, carry):
        for b in range(DBUF):
            j = t * DBUF + b
            wait_gathers(j, b)

            @pl.when(t > 0)
            def _():
                wait_out(b)

            compute(b, j)
            store_out(j, b)

            @pl.when(j + DBUF < CHUNKS_W)
            def _():
                gathers(j + DBUF, b)
        return carry

    lax.fori_loop(0, DGRP, group, 0)
    for b in range(CHUNKS_W - DBUF * DGRP):
        j = DBUF * DGRP + b
        wait_gathers(j, b)
        wait_out(b)
        compute(b, j)
        store_out(j, b)
    for b in range(DBUF):
        wait_out(b)


_sc_decode = pl.kernel(
    _sc_decode_body,
    out_type=jax.ShapeDtypeStruct((E_PAD,), jnp.float32),
    mesh=_mesh,
    compiler_params=pltpu.CompilerParams(needs_layout_passes=False),
    scratch_types=[
        pltpu.VMEM((CHUNKS_W, CHUNK), jnp.int32),
        pltpu.VMEM((CHUNKS_W, CHUNK), jnp.int32),
        pltpu.VMEM((DBUF, CHUNK, D), jnp.float32),
        pltpu.VMEM((DBUF, CHUNK, D), jnp.float32),
        pltpu.VMEM((CHUNK // 16 * 256,), jnp.float32),
        pltpu.VMEM((DBUF, CHUNK), jnp.float32),
        pltpu.SemaphoreType.DMA((DBUF,)),
        pltpu.SemaphoreType.DMA((DBUF,)),
    ],
)


ROW_BLK = 1024
GRID = N_PAD // ROW_BLK


def _tc_mm_scale_body(deg0_ref, deg1_ref, x_ref, w_ref, g_ref, dis_ref):
    deg = deg0_ref[...] + deg1_ref[...] + 1.0
    dis = lax.rsqrt(deg)
    m = jnp.dot(x_ref[...], w_ref[...], preferred_element_type=jnp.float32)
    g_ref[...] = m * dis
    dis_ref[...] = dis


def _tc_mm_scale(deg0, deg1, x, w):
    return pl.pallas_call(
        _tc_mm_scale_body,
        grid=(GRID,),
        in_specs=[
            pl.BlockSpec((ROW_BLK, 1), lambda i: (i, 0)),
            pl.BlockSpec((ROW_BLK, 1), lambda i: (i, 0)),
            pl.BlockSpec((ROW_BLK, D), lambda i: (i, 0)),
            pl.BlockSpec((D, D), lambda i: (0, 0)),
        ],
        out_specs=[
            pl.BlockSpec((ROW_BLK, D), lambda i: (i, 0)),
            pl.BlockSpec((ROW_BLK, 1), lambda i: (i, 0)),
        ],
        out_shape=[
            jax.ShapeDtypeStruct((N_PAD, D), jnp.float32),
            jax.ShapeDtypeStruct((N_PAD, 1), jnp.float32),
        ],
    )(deg0, deg1, x, w)


def _tc_mid_body(p0_ref, p1_ref, gp_ref, dis_ref, b_ref, w_ref, g_ref):
    dis = dis_ref[...]
    agg = (p0_ref[...] + p1_ref[...] - gp_ref[...]) * dis
    h = jnp.maximum(agg + b_ref[...], 0.0)
    g_ref[...] = jnp.dot(h, w_ref[...], preferred_element_type=jnp.float32) * dis


def _tc_mid(p0, p1, gp, dis, b, w):
    return pl.pallas_call(
        _tc_mid_body,
        grid=(GRID,),
        in_specs=[
            pl.BlockSpec((ROW_BLK, D), lambda i: (i, 0)),
            pl.BlockSpec((ROW_BLK, D), lambda i: (i, 0)),
            pl.BlockSpec((ROW_BLK, D), lambda i: (i, 0)),
            pl.BlockSpec((ROW_BLK, 1), lambda i: (i, 0)),
            pl.BlockSpec((1, D), lambda i: (0, 0)),
            pl.BlockSpec((D, D), lambda i: (0, 0)),
        ],
        out_specs=pl.BlockSpec((ROW_BLK, D), lambda i: (i, 0)),
        out_shape=jax.ShapeDtypeStruct((N_PAD, D), jnp.float32),
    )(p0, p1, gp, dis, b, w)


def _tc_z_body(p0_ref, p1_ref, gp_ref, dis_ref, b_ref, z_ref):
    z_ref[...] = ((p0_ref[...] + p1_ref[...] - gp_ref[...]) * dis_ref[...]
                  + b_ref[...])


def _tc_z(p0, p1, gp, dis, b):
    return pl.pallas_call(
        _tc_z_body,
        grid=(GRID,),
        in_specs=[
            pl.BlockSpec((ROW_BLK, D), lambda i: (i, 0)),
            pl.BlockSpec((ROW_BLK, D), lambda i: (i, 0)),
            pl.BlockSpec((ROW_BLK, D), lambda i: (i, 0)),
            pl.BlockSpec((ROW_BLK, 1), lambda i: (i, 0)),
            pl.BlockSpec((1, D), lambda i: (0, 0)),
        ],
        out_specs=pl.BlockSpec((ROW_BLK, D), lambda i: (i, 0)),
        out_shape=jax.ShapeDtypeStruct((N_PAD, D), jnp.float32),
    )(p0, p1, gp, dis, b)


def kernel(x, edge_index, W1, b1, W2, b2, W3, b3):
    n_pad_e = E_PAD - N_EDGES
    pad_idx = N_NODES + (jnp.arange(n_pad_e, dtype=jnp.int32)
                         % (N_PAD - N_NODES))
    src = jnp.concatenate([edge_index[0], pad_idx])
    dst = jnp.concatenate([edge_index[1], pad_idx])
    src2 = src.reshape(N_CHUNKS, CHUNK)
    dst2 = dst.reshape(N_CHUNKS, CHUNK)
    dst3 = dst.reshape(N_CHUNKS, 2, 64)

    degp = _sc_deg(dst2)
    deg0 = degp[0].reshape(N_PAD, 1)
    deg1 = degp[1].reshape(N_PAD, 1)

    xp = jnp.zeros((N_PAD, D), jnp.float32).at[:N_NODES].set(x)
    g1, dis = _tc_mm_scale(deg0, deg1, xp, W1)
    p1 = _sc_layer(g1, src, dst3)
    g2 = _tc_mid(p1[0], p1[1], g1, dis, b1.reshape(1, D), W2)
    p2 = _sc_layer(g2, src, dst3)
    g3 = _tc_mid(p2[0], p2[1], g2, dis, b2.reshape(1, D), W3)
    p3 = _sc_layer(g3, src, dst3)
    z = _tc_z(p3[0], p3[1], g3, dis, b3.reshape(1, D))
    return _sc_decode(z, src2, dst2)[:N_EDGES]
